# Initial kernel scaffold; baseline (speedup 1.0000x reference)
import jax
import jax.numpy as jnp
from jax.experimental import pallas as pl


def kernel(seg_out, det_out, det_scores, det_indices):
    raise NotImplementedError("write your pallas kernel here")


# speedup vs baseline: 1.1558x; 1.1558x over previous
import functools

import jax
import jax.numpy as jnp
from jax import lax
from jax.experimental import pallas as pl
from jax.experimental.pallas import tpu as pltpu
from jax.experimental.pallas import tpu_sc as plsc

B, N, D, M = 16, 1024, 768, 300
MP = 320
NW = 32
PW = B * MP // NW
CH = 80
LANES = 16


def _sc_gather(seg_flat, idx_flat):
    mesh = plsc.VectorSubcoreMesh(core_axis_name="c", subcore_axis_name="s")

    @functools.partial(
        pl.kernel,
        mesh=mesh,
        out_type=jax.ShapeDtypeStruct((B * MP, D), jnp.float32),
        scratch_types=[
            pltpu.VMEM((PW,), jnp.int32),
            pltpu.VMEM((PW, D), jnp.float32),
            pltpu.SemaphoreType.DMA,
        ],
    )
    def k(seg_hbm, idx_hbm, out_hbm, idx_v, rows_v, sem):
        wid = lax.axis_index("s") * 2 + lax.axis_index("c")
        base = wid * PW
        row_off = (wid // 2) * N

        pltpu.sync_copy(idx_hbm.at[pl.ds(base, PW)], idx_v)
        for j in range(PW // LANES):
            sl = pl.ds(j * LANES, LANES)
            idx_v[sl] = idx_v[sl] + row_off
        for c in range(PW // CH):
            pltpu.async_copy(
                seg_hbm.at[idx_v.at[pl.ds(c * CH, CH)]],
                rows_v.at[pl.ds(c * CH, CH)],
                sem,
            ).wait()
        pltpu.sync_copy(rows_v, out_hbm.at[pl.ds(base, PW)])

    return k(seg_flat, idx_flat)


def kernel(seg_out, det_out, det_scores, det_indices):
    idx = det_indices.astype(jnp.int32)
    idx = jnp.pad(idx, ((0, 0), (0, MP - M)))
    out = _sc_gather(seg_out.reshape(B * N, D), idx.reshape(B * MP))
    return out.reshape(B, MP, D)[:, :M, :]


# speedup vs baseline: 1.1918x; 1.0312x over previous
import functools

import jax
import jax.numpy as jnp
from jax import lax
from jax.experimental import pallas as pl
from jax.experimental.pallas import tpu as pltpu
from jax.experimental.pallas import tpu_sc as plsc

B, N, D, M = 16, 1024, 768, 300
NW_ACT = 30
PW = B * M // NW_ACT
CH = 80
LANES = 16


def _sc_gather(seg_flat, idx_flat):
    mesh = plsc.VectorSubcoreMesh(core_axis_name="c", subcore_axis_name="s")

    @functools.partial(
        pl.kernel,
        mesh=mesh,
        out_type=jax.ShapeDtypeStruct((B * M, D), jnp.float32),
        scratch_types=[
            pltpu.VMEM((PW,), jnp.int32),
            pltpu.VMEM((PW, D), jnp.float32),
            pltpu.SemaphoreType.DMA,
        ],
    )
    def k(seg_hbm, idx_hbm, out_hbm, idx_v, rows_v, sem):
        wid = lax.axis_index("s") * 2 + lax.axis_index("c")

        @pl.when(wid < NW_ACT)
        def _():
            base = wid * PW
            pltpu.sync_copy(idx_hbm.at[pl.ds(base, PW)], idx_v)
            iot = lax.iota(jnp.int32, 16)
            b0 = base // M
            boundary = (b0 + 1) * M
            for j in range(PW // LANES):
                sl = pl.ds(j * LANES, LANES)
                pos = base + j * LANES + iot
                bump = jnp.where(pos >= boundary, N, 0)
                idx_v[sl] = idx_v[sl] + (b0 * N + bump)
            for c in range(PW // CH):
                pltpu.async_copy(
                    seg_hbm.at[idx_v.at[pl.ds(c * CH, CH)]],
                    rows_v.at[pl.ds(c * CH, CH)],
                    sem,
                ).wait()
            pltpu.sync_copy(rows_v, out_hbm.at[pl.ds(base, PW)])

    return k(seg_flat, idx_flat)


def kernel(seg_out, det_out, det_scores, det_indices):
    idx = det_indices.astype(jnp.int32).reshape(B * M)
    out = _sc_gather(seg_out.reshape(B * N, D), idx)
    return out.reshape(B, M, D)


# speedup vs baseline: 1.4380x; 1.2066x over previous
import functools

import jax
import jax.numpy as jnp
from jax import lax
from jax.experimental import pallas as pl
from jax.experimental.pallas import tpu as pltpu
from jax.experimental.pallas import tpu_sc as plsc

B, N, D, M = 16, 1024, 768, 300
MP = 304
PW0 = 160
PW1 = MP - PW0
LANES = 16


def _sc_gather(seg_flat, idx_flat):
    mesh = plsc.VectorSubcoreMesh(core_axis_name="c", subcore_axis_name="s")

    @functools.partial(
        pl.kernel,
        mesh=mesh,
        out_type=jax.ShapeDtypeStruct((B, MP, D), jnp.float32),
        scratch_types=[
            pltpu.VMEM((PW0,), jnp.int32),
            pltpu.VMEM((PW0, D), jnp.float32),
            pltpu.SemaphoreType.DMA,
        ],
    )
    def k(seg_hbm, idx_hbm, out_hbm, idx_v, rows_v, sem):
        wid = lax.axis_index("s") * 2 + lax.axis_index("c")
        b = wid // 2
        half = wid % 2
        row_off = b * N

        @pl.when(half == 0)
        def _():
            pltpu.sync_copy(idx_hbm.at[pl.ds(b * MP, PW0)], idx_v)
            for j in range(PW0 // LANES):
                sl = pl.ds(j * LANES, LANES)
                idx_v[sl] = idx_v[sl] + row_off
            for c in range(2):
                pltpu.async_copy(
                    seg_hbm.at[idx_v.at[pl.ds(c * 80, 80)]],
                    rows_v.at[pl.ds(c * 80, 80)],
                    sem,
                ).wait()
            pltpu.sync_copy(rows_v, out_hbm.at[b, pl.ds(0, PW0), :])

        @pl.when(half == 1)
        def _():
            pltpu.sync_copy(idx_hbm.at[pl.ds(b * MP + PW0, PW1)], idx_v.at[pl.ds(0, PW1)])
            for j in range(PW1 // LANES):
                sl = pl.ds(j * LANES, LANES)
                idx_v[sl] = idx_v[sl] + row_off
            for c in range(2):
                pltpu.async_copy(
                    seg_hbm.at[idx_v.at[pl.ds(c * 72, 72)]],
                    rows_v.at[pl.ds(c * 72, 72)],
                    sem,
                ).wait()
            pltpu.sync_copy(
                rows_v.at[pl.ds(0, PW1)], out_hbm.at[b, pl.ds(PW0, PW1), :]
            )

    return k(seg_flat, idx_flat)


def kernel(seg_out, det_out, det_scores, det_indices):
    idx = det_indices.astype(jnp.int32)
    idx = jnp.pad(idx, ((0, 0), (0, MP - M)))
    out = _sc_gather(seg_out.reshape(B * N, D), idx.reshape(B * MP))
    return out[:, :M, :]


# speedup vs baseline: 1.4478x; 1.0068x over previous
import functools

import jax
import jax.numpy as jnp
from jax import lax
from jax.experimental import pallas as pl
from jax.experimental.pallas import tpu as pltpu
from jax.experimental.pallas import tpu_sc as plsc

B, N, D, M = 16, 1024, 768, 300
MP = 304
PW0 = 160
PW1 = 136
MT = 296
LANES = 16


def _sc_gather(seg_flat, idx_flat):
    mesh = plsc.VectorSubcoreMesh(core_axis_name="c", subcore_axis_name="s")

    @functools.partial(
        pl.kernel,
        mesh=mesh,
        out_type=jax.ShapeDtypeStruct((B, M, D), jnp.float32),
        scratch_types=[
            pltpu.VMEM((PW0,), jnp.int32),
            pltpu.VMEM((PW0, D), jnp.float32),
            pltpu.SemaphoreType.DMA,
        ],
    )
    def k(seg_hbm, idx_hbm, out_hbm, idx_v, rows_v, sem):
        wid = lax.axis_index("s") * 2 + lax.axis_index("c")
        b = wid // 2
        half = wid % 2
        row_off = b * N

        @pl.when(half == 0)
        def _():
            pltpu.sync_copy(idx_hbm.at[pl.ds(b * MP, PW0)], idx_v)
            for j in range(PW0 // LANES):
                sl = pl.ds(j * LANES, LANES)
                idx_v[sl] = idx_v[sl] + row_off
            for c in range(2):
                pltpu.async_copy(
                    seg_hbm.at[idx_v.at[pl.ds(c * 80, 80)]],
                    rows_v.at[pl.ds(c * 80, 80)],
                    sem,
                ).wait()
            pltpu.sync_copy(rows_v, out_hbm.at[b, pl.ds(0, PW0), :])

        @pl.when(half == 1)
        def _():
            pltpu.sync_copy(
                idx_hbm.at[pl.ds(b * MP + PW0, 144)], idx_v.at[pl.ds(0, 144)]
            )
            for j in range(144 // LANES):
                sl = pl.ds(j * LANES, LANES)
                idx_v[sl] = idx_v[sl] + row_off
            pltpu.async_copy(
                seg_hbm.at[idx_v.at[pl.ds(0, 72)]],
                rows_v.at[pl.ds(0, 72)],
                sem,
            ).wait()
            pltpu.async_copy(
                seg_hbm.at[idx_v.at[pl.ds(72, 64)]],
                rows_v.at[pl.ds(72, 64)],
                sem,
            ).wait()
            pltpu.sync_copy(
                rows_v.at[pl.ds(0, PW1)], out_hbm.at[b, pl.ds(PW0, PW1), :]
            )

    return k(seg_flat, idx_flat)


def kernel(seg_out, det_out, det_scores, det_indices):
    idx = det_indices.astype(jnp.int32)
    idx_padded = jnp.pad(idx, ((0, 0), (0, MP - M)))
    out = _sc_gather(seg_out.reshape(B * N, D), idx_padded.reshape(B * MP))
    tail = jnp.take_along_axis(seg_out, idx[:, MT:M, None], axis=1)
    return lax.dynamic_update_slice(out, tail, (0, MT, 0))
